# baseline (speedup 1.0000x reference)
import functools

import jax
import jax.numpy as jnp
import numpy as np
from jax.experimental import pallas as pl
from jax.experimental.pallas import tpu as pltpu

_N_CLASSES = 128
_N_BINS = 10
_BOUNDS = [np.float32(v) for v in np.linspace(0.0, 1.0, _N_BINS + 1)[:-1]]


def _bf16_threshold(u):
    bits = int(np.asarray(u, dtype=np.float32).astype(jnp.bfloat16)
               .view(np.uint16))
    best_t, best_d = None, None
    for cb in (bits - 2, bits - 1, bits, bits + 1):
        t = np.asarray(cb, dtype=np.uint16).view(jnp.bfloat16)
        tn = np.asarray(cb + 1, dtype=np.uint16).view(jnp.bfloat16)
        boundary = (float(t) + float(tn)) / 2.0
        d = abs(boundary - float(u))
        if best_d is None or d < best_d:
            best_t, best_d = t, d
    return best_t


_BF_THRESH = [_bf16_threshold(u) for u in _BOUNDS[1:]]

_ROWS = 8192
_CHUNK = 2048


def _hist_kernel(x_ref, t_ref, loss_ref, acc_ref, *, n_steps, total):
    step = pl.program_id(0)

    ones = jnp.ones((8, _CHUNK), jnp.bfloat16)
    zero_b = jnp.bfloat16(0)
    cls = jax.lax.broadcasted_iota(jnp.int32, (_CHUNK, _N_CLASSES), 1)

    dots = [jnp.zeros((8, _N_CLASSES), jnp.float32) for _ in range(_N_BINS)]
    for c0 in range(0, _ROWS, _CHUNK):
        x = x_ref[c0:c0 + _CHUNK, :]
        t = t_ref[c0:c0 + _CHUNK, :]
        m = jnp.max(x, axis=1, keepdims=True)
        e = jnp.exp(x - m)
        s = jnp.sum(e, axis=1, keepdims=True)
        r = pl.reciprocal(s, approx=True)
        p = e * r

        gt = t == cls
        gtf = jnp.where(gt, 1.0, 0.0)
        q = gtf - p

        pb = p.astype(jnp.bfloat16)
        qb = q.astype(jnp.bfloat16)

        sels = [qb]
        for tb in _BF_THRESH:
            sels.append(jnp.where(pb > tb, qb, zero_b))
        for i, sel in enumerate(sels):
            dots[i] = dots[i] + jnp.dot(
                ones, sel, preferred_element_type=jnp.float32)

    @pl.when(step == 0)
    def _():
        for i in range(_N_BINS):
            acc_ref[8 * i:8 * i + 8, :] = dots[i]

    @pl.when(step > 0)
    def _():
        for i in range(_N_BINS):
            acc_ref[8 * i:8 * i + 8, :] += dots[i]

    @pl.when(step == n_steps - 1)
    def _():
        a = acc_ref[...]
        d_cum = jnp.concatenate(
            [a[8 * i:8 * i + 1] for i in range(_N_BINS)], axis=0)
        d_next = jnp.concatenate(
            [d_cum[1:], jnp.zeros((1, _N_CLASSES), jnp.float32)], axis=0)
        per_bin = d_cum - d_next
        loss_ref[0, 0] = jnp.sum(jnp.abs(per_bin)) / total


def kernel(output, target):
    batch, n_classes = output.shape
    n_steps = batch // _ROWS
    t2 = target.reshape(batch, 1)

    loss = pl.pallas_call(
        functools.partial(_hist_kernel, n_steps=n_steps,
                          total=float(batch * n_classes)),
        grid=(n_steps,),
        in_specs=[
            pl.BlockSpec((_ROWS, n_classes), lambda i: (i, 0)),
            pl.BlockSpec((_ROWS, 1), lambda i: (i, 0)),
        ],
        out_specs=pl.BlockSpec((1, 1), lambda i: (0, 0), memory_space=pltpu.SMEM),
        out_shape=jax.ShapeDtypeStruct((1, 1), jnp.float32),
        scratch_shapes=[pltpu.VMEM((80, _N_CLASSES), jnp.float32)],
    )(output, t2)
    return loss[0, 0]


# speedup vs baseline: 1.1322x; 1.1322x over previous
import functools

import jax
import jax.numpy as jnp
import numpy as np
from jax.experimental import pallas as pl
from jax.experimental.pallas import tpu as pltpu

_N_CLASSES = 128
_N_BINS = 10
_BOUNDS = [np.float32(v) for v in np.linspace(0.0, 1.0, _N_BINS + 1)[:-1]]


def _bf16_threshold(u):
    bits = int(np.asarray(u, dtype=np.float32).astype(jnp.bfloat16)
               .view(np.uint16))
    best_t, best_d = None, None
    for cb in (bits - 2, bits - 1, bits, bits + 1):
        t = np.asarray(cb, dtype=np.uint16).view(jnp.bfloat16)
        tn = np.asarray(cb + 1, dtype=np.uint16).view(jnp.bfloat16)
        boundary = (float(t) + float(tn)) / 2.0
        d = abs(boundary - float(u))
        if best_d is None or d < best_d:
            best_t, best_d = t, d
    return best_t


_BF_THRESH = [_bf16_threshold(u) for u in _BOUNDS[1:]]

_ROWS = 16384
_CHUNK = 1024


def _hist_kernel(x_ref, t_ref, loss_ref, acc_ref, *, n_steps, total):
    step = pl.program_id(0)

    ones = jnp.ones((8, _CHUNK), jnp.bfloat16)
    zero_b = jnp.bfloat16(0)
    cls = jax.lax.broadcasted_iota(jnp.int32, (_CHUNK, _N_CLASSES), 1)

    dots = [jnp.zeros((8, _N_CLASSES), jnp.float32) for _ in range(_N_BINS)]
    for c0 in range(0, _ROWS, _CHUNK):
        x = x_ref[c0:c0 + _CHUNK, :]
        t = t_ref[c0:c0 + _CHUNK, :]
        m = jnp.max(x, axis=1, keepdims=True)
        e = jnp.exp(x - m)
        s = jnp.sum(e, axis=1, keepdims=True)
        r = pl.reciprocal(s, approx=True)
        p = e * r

        gt = t == cls
        gtf = jnp.where(gt, 1.0, 0.0)
        q = gtf - p

        pb = p.astype(jnp.bfloat16)
        qb = q.astype(jnp.bfloat16)

        sels = [qb]
        for tb in _BF_THRESH:
            sels.append(jnp.where(pb > tb, qb, zero_b))
        for i, sel in enumerate(sels):
            dots[i] = dots[i] + jnp.dot(
                ones, sel, preferred_element_type=jnp.float32)

    @pl.when(step == 0)
    def _():
        for i in range(_N_BINS):
            acc_ref[8 * i:8 * i + 8, :] = dots[i]

    @pl.when(step > 0)
    def _():
        for i in range(_N_BINS):
            acc_ref[8 * i:8 * i + 8, :] += dots[i]

    @pl.when(step == n_steps - 1)
    def _():
        a = acc_ref[...]
        d_cum = jnp.concatenate(
            [a[8 * i:8 * i + 1] for i in range(_N_BINS)], axis=0)
        d_next = jnp.concatenate(
            [d_cum[1:], jnp.zeros((1, _N_CLASSES), jnp.float32)], axis=0)
        per_bin = d_cum - d_next
        loss_ref[0, 0] = jnp.sum(jnp.abs(per_bin)) / total


def kernel(output, target):
    batch, n_classes = output.shape
    n_steps = batch // _ROWS
    t2 = target.reshape(batch, 1)

    loss = pl.pallas_call(
        functools.partial(_hist_kernel, n_steps=n_steps,
                          total=float(batch * n_classes)),
        grid=(n_steps,),
        in_specs=[
            pl.BlockSpec((_ROWS, n_classes), lambda i: (i, 0)),
            pl.BlockSpec((_ROWS, 1), lambda i: (i, 0)),
        ],
        out_specs=pl.BlockSpec((1, 1), lambda i: (0, 0), memory_space=pltpu.SMEM),
        out_shape=jax.ShapeDtypeStruct((1, 1), jnp.float32),
        scratch_shapes=[pltpu.VMEM((80, _N_CLASSES), jnp.float32)],
    )(output, t2)
    return loss[0, 0]


# speedup vs baseline: 1.1795x; 1.0418x over previous
import functools

import jax
import jax.numpy as jnp
import numpy as np
from jax.experimental import pallas as pl
from jax.experimental.pallas import tpu as pltpu

_N_CLASSES = 128
_N_BINS = 10
_BOUNDS = [np.float32(v) for v in np.linspace(0.0, 1.0, _N_BINS + 1)[:-1]]


def _bf16_threshold(u):
    bits = int(np.asarray(u, dtype=np.float32).astype(jnp.bfloat16)
               .view(np.uint16))
    best_t, best_d = None, None
    for cb in (bits - 2, bits - 1, bits, bits + 1):
        t = np.asarray(cb, dtype=np.uint16).view(jnp.bfloat16)
        tn = np.asarray(cb + 1, dtype=np.uint16).view(jnp.bfloat16)
        boundary = (float(t) + float(tn)) / 2.0
        d = abs(boundary - float(u))
        if best_d is None or d < best_d:
            best_t, best_d = t, d
    return best_t


_BF_THRESH = [_bf16_threshold(u) for u in _BOUNDS[1:]]

_ROWS = 8192
_CHUNK = 512


def _hist_kernel(x_ref, t_ref, loss_ref, acc_ref, *, n_steps, total):
    step = pl.program_id(0)

    ones = jnp.ones((8, _CHUNK), jnp.bfloat16)
    zero_b = jnp.bfloat16(0)
    cls = jax.lax.broadcasted_iota(jnp.int32, (_CHUNK, _N_CLASSES), 1)

    dots = [jnp.zeros((8, _N_CLASSES), jnp.float32) for _ in range(_N_BINS)]
    for c0 in range(0, _ROWS, _CHUNK):
        x = x_ref[c0:c0 + _CHUNK, :]
        t = t_ref[c0:c0 + _CHUNK, :]
        m = jnp.max(x, axis=1, keepdims=True)
        e = jnp.exp(x - m)
        s = jnp.sum(e, axis=1, keepdims=True)
        r = pl.reciprocal(s, approx=True)
        p = e * r

        gt = t == cls
        gtf = jnp.where(gt, 1.0, 0.0)
        q = gtf - p

        pb = p.astype(jnp.bfloat16)
        qb = q.astype(jnp.bfloat16)

        sels = [qb]
        for tb in _BF_THRESH:
            sels.append(jnp.where(pb > tb, qb, zero_b))
        for i, sel in enumerate(sels):
            dots[i] = dots[i] + jnp.dot(
                ones, sel, preferred_element_type=jnp.float32)

    @pl.when(step == 0)
    def _():
        for i in range(_N_BINS):
            acc_ref[8 * i:8 * i + 8, :] = dots[i]

    @pl.when(step > 0)
    def _():
        for i in range(_N_BINS):
            acc_ref[8 * i:8 * i + 8, :] += dots[i]

    @pl.when(step == n_steps - 1)
    def _():
        a = acc_ref[...]
        d_cum = jnp.concatenate(
            [a[8 * i:8 * i + 1] for i in range(_N_BINS)], axis=0)
        d_next = jnp.concatenate(
            [d_cum[1:], jnp.zeros((1, _N_CLASSES), jnp.float32)], axis=0)
        per_bin = d_cum - d_next
        loss_ref[0, 0] = jnp.sum(jnp.abs(per_bin)) / total


def kernel(output, target):
    batch, n_classes = output.shape
    n_steps = batch // _ROWS
    t2 = target.reshape(batch, 1)

    loss = pl.pallas_call(
        functools.partial(_hist_kernel, n_steps=n_steps,
                          total=float(batch * n_classes)),
        grid=(n_steps,),
        in_specs=[
            pl.BlockSpec((_ROWS, n_classes), lambda i: (i, 0)),
            pl.BlockSpec((_ROWS, 1), lambda i: (i, 0)),
        ],
        out_specs=pl.BlockSpec((1, 1), lambda i: (0, 0), memory_space=pltpu.SMEM),
        out_shape=jax.ShapeDtypeStruct((1, 1), jnp.float32),
        scratch_shapes=[pltpu.VMEM((80, _N_CLASSES), jnp.float32)],
    )(output, t2)
    return loss[0, 0]
